# Initial kernel scaffold; baseline (speedup 1.0000x reference)
import jax
import jax.numpy as jnp
from jax.experimental import pallas as pl


def kernel(edge_index, edge_type, concat_emb, w1, b1, w2, b2, rw1, rr1, rb1, rw2, rr2, rb2):
    raise NotImplementedError("write your pallas kernel here")


# speedup vs baseline: 15.2322x; 15.2322x over previous
import functools

import jax
import jax.numpy as jnp
from jax import lax
from jax.experimental import pallas as pl
from jax.experimental.pallas import tpu as pltpu
from jax.experimental.pallas import tpu_sc as plsc

N = 10000
E = 320000
IN_F = 384
OUT_F = 256
EMB = 128
HID = 64
LAB = 32
R = 8
NR = N * R

L = 16
NC = 2
NS = 16
NW = NC * NS
EW = E // NW
B = 80
NCHUNK = EW // B

NB = 400
NBLK = N // NB


def _tc_a_body(emb, w1t, b1, w2t, b2, rw1c, rr1, rb1, y1, root1):
    x1 = jnp.tanh(jnp.dot(emb[...], w1t[...],
                          preferred_element_type=jnp.float32) + b1[...])
    x2 = jnp.dot(x1, w2t[...], preferred_element_type=jnp.float32) + b2[...]
    y1[...] = jnp.dot(x2, rw1c[...], preferred_element_type=jnp.float32)
    root1[...] = jnp.dot(x2, rr1[...],
                         preferred_element_type=jnp.float32) + rb1[...]


def _tc_b_body(root1, p1, rw2c, rr2, rb2, y2, root2):
    h = jnp.maximum(root1[...] + p1[0] + p1[1], 0.0)
    y2[...] = jnp.dot(h, rw2c[...], preferred_element_type=jnp.float32)
    root2[...] = jnp.dot(h, rr2[...],
                         preferred_element_type=jnp.float32) + rb2[...]


def _tc_f_body(root2, p2, out):
    out[...] = jax.nn.sigmoid(root2[...] + p2[0] + p2[1])


def _full(shape):
    return pl.BlockSpec(shape, lambda i: tuple(0 for _ in shape))


def _tc_a(emb, w1t, b1, w2t, b2, rw1c, rr1, rb1):
    return pl.pallas_call(
        _tc_a_body,
        grid=(NBLK,),
        in_specs=[
            pl.BlockSpec((NB, IN_F), lambda i: (i, 0)),
            _full((IN_F, OUT_F)),
            _full((1, OUT_F)),
            _full((OUT_F, EMB)),
            _full((1, EMB)),
            _full((EMB, R * HID)),
            _full((EMB, HID)),
            _full((1, HID)),
        ],
        out_specs=[
            pl.BlockSpec((NB, R * HID), lambda i: (i, 0)),
            pl.BlockSpec((NB, HID), lambda i: (i, 0)),
        ],
        out_shape=[
            jax.ShapeDtypeStruct((N, R * HID), jnp.float32),
            jax.ShapeDtypeStruct((N, HID), jnp.float32),
        ],
    )(emb, w1t, b1, w2t, b2, rw1c, rr1, rb1)


def _tc_b(root1, p1, rw2c, rr2, rb2):
    return pl.pallas_call(
        _tc_b_body,
        grid=(NBLK,),
        in_specs=[
            pl.BlockSpec((NB, HID), lambda i: (i, 0)),
            pl.BlockSpec((NC, NB, HID), lambda i: (0, i, 0)),
            _full((HID, R * LAB)),
            _full((HID, LAB)),
            _full((1, LAB)),
        ],
        out_specs=[
            pl.BlockSpec((NB, R * LAB), lambda i: (i, 0)),
            pl.BlockSpec((NB, LAB), lambda i: (i, 0)),
        ],
        out_shape=[
            jax.ShapeDtypeStruct((N, R * LAB), jnp.float32),
            jax.ShapeDtypeStruct((N, LAB), jnp.float32),
        ],
    )(root1, p1, rw2c, rr2, rb2)


def _tc_f(root2, p2):
    return pl.pallas_call(
        _tc_f_body,
        grid=(NBLK,),
        in_specs=[
            pl.BlockSpec((NB, LAB), lambda i: (i, 0)),
            pl.BlockSpec((NC, NB, LAB), lambda i: (0, i, 0)),
        ],
        out_specs=pl.BlockSpec((NB, LAB), lambda i: (i, 0)),
        out_shape=jax.ShapeDtypeStruct((N, LAB), jnp.float32),
    )(root2, p2)


def _zrow():
    return jnp.zeros((L,), jnp.float32)


def _onerow():
    return jnp.ones((L,), jnp.float32)


def _splat(v16, lane):
    idx = jnp.full((L, 1), lane, jnp.int32)
    return lax.gather(
        v16, idx,
        lax.GatherDimensionNumbers(
            offset_dims=(), collapsed_slice_dims=(0,), start_index_map=(0,)),
        slice_sizes=(1,),
        mode=lax.GatherScatterMode.PROMISE_IN_BOUNDS)


def _sc_counts_kernel(dst_hbm, typ_hbm, cnt_hbm, d_v, t_v, wi_v, ones_v,
                      zb_v, sem, acc_sp):
    cid = lax.axis_index("c")
    sid = lax.axis_index("s")
    wid = sid * NC + cid

    def _fill(i, _):
        ones_v[i, :] = _onerow()
        zb_v[i, :] = _zrow()
        return _
    lax.fori_loop(0, B, _fill, None)

    def _zfill(i, _):
        zb_v[i, :] = _zrow()
        return _
    lax.fori_loop(B, 1000, _zfill, None)

    def _zero(j, _):
        pltpu.sync_copy(zb_v, acc_sp.at[pl.ds(sid * (NR // NS) + j * 1000,
                                              1000)])
        return _
    lax.fori_loop(0, NR // NS // 1000, _zero, None)
    plsc.subcore_barrier()

    def _chunk(i, _):
        base = wid * EW + i * B
        pltpu.sync_copy(dst_hbm.at[pl.ds(base, B)], d_v)
        pltpu.sync_copy(typ_hbm.at[pl.ds(base, B)], t_v)
        for g in range(B // L):
            sl = pl.ds(g * L, L)
            wi_v[sl] = d_v[sl] * R + t_v[sl]
        pltpu.sync_copy(ones_v, acc_sp.at[wi_v], add=True)
        return _
    lax.fori_loop(0, NCHUNK, _chunk, None)
    plsc.subcore_barrier()

    pltpu.sync_copy(acc_sp.at[pl.ds(sid * (NR // NS), NR // NS)],
                    cnt_hbm.at[cid, pl.ds(sid * (NR // NS), NR // NS)])


def _sc_counts(dstv, typv):
    kfn = functools.partial(
        pl.kernel,
        out_type=jax.ShapeDtypeStruct((NC, NR, L), jnp.float32),
        mesh=plsc.VectorSubcoreMesh(core_axis_name="c", subcore_axis_name="s"),
        compiler_params=pltpu.CompilerParams(use_tc_tiling_on_sc=False, needs_layout_passes=False),
        scratch_types=[
            pltpu.VMEM((B,), jnp.int32),
            pltpu.VMEM((B,), jnp.int32),
            pltpu.VMEM((B,), jnp.int32),
            pltpu.VMEM((B, L), jnp.float32),
            pltpu.VMEM((1000, L), jnp.float32),
            pltpu.SemaphoreType.DMA,
            pltpu.VMEM_SHARED((NR, L), jnp.float32),
        ],
    )(_sc_counts_kernel)
    return kfn(dstv, typv)


def _make_sc_agg_kernel(h):
    nsl = h // L
    rows_per_sub = N // NS
    zrows = 25

    def body(ytab_hbm, inv_hbm, src_hbm, dst_hbm, typ_hbm, part_hbm,
             s_v, d_v, t_v, g_v, wi_v, rows_v, inv_v, zb_v, sem, acc_sp):
        cid = lax.axis_index("c")
        sid = lax.axis_index("s")
        wid = sid * NC + cid

        pltpu.sync_copy(inv_hbm, inv_v)

        def _zfill(i, _):
            for k in range(nsl):
                zb_v[i, pl.ds(k * L, L)] = _zrow()
            return _
        lax.fori_loop(0, zrows, _zfill, None)

        def _zero(j, _):
            pltpu.sync_copy(zb_v,
                            acc_sp.at[pl.ds(sid * rows_per_sub + j * zrows,
                                            zrows)])
            return _
        lax.fori_loop(0, rows_per_sub // zrows, _zero, None)
        plsc.subcore_barrier()

        def _chunk(i, _):
            base = wid * EW + i * B
            pltpu.sync_copy(src_hbm.at[pl.ds(base, B)], s_v)
            pltpu.sync_copy(dst_hbm.at[pl.ds(base, B)], d_v)
            pltpu.sync_copy(typ_hbm.at[pl.ds(base, B)], t_v)
            for g in range(B // L):
                sl = pl.ds(g * L, L)
                t16 = t_v[sl]
                g_v[sl] = s_v[sl] * R + t16
                wi_v[sl] = d_v[sl] * R + t16
            pltpu.async_copy(ytab_hbm.at[g_v], rows_v, sem).wait()
            for g in range(B // L):
                w16 = plsc.load_gather(inv_v, [wi_v[pl.ds(g * L, L)]])
                for j in range(L):
                    e = g * L + j
                    ws = _splat(w16, j)
                    for k in range(nsl):
                        sl = pl.ds(k * L, L)
                        rows_v[e, sl] = rows_v[e, sl] * ws
            pltpu.sync_copy(rows_v, acc_sp.at[d_v], add=True)
            return _
        lax.fori_loop(0, NCHUNK, _chunk, None)
        plsc.subcore_barrier()

        pltpu.sync_copy(acc_sp.at[pl.ds(sid * rows_per_sub, rows_per_sub)],
                        part_hbm.at[cid, pl.ds(sid * rows_per_sub,
                                               rows_per_sub)])

    def run(ytab, inv, srcv, dstv, typv):
        kfn = functools.partial(
            pl.kernel,
            out_type=jax.ShapeDtypeStruct((NC, N, h), jnp.float32),
            mesh=plsc.VectorSubcoreMesh(core_axis_name="c",
                                        subcore_axis_name="s"),
            compiler_params=pltpu.CompilerParams(use_tc_tiling_on_sc=False, needs_layout_passes=False),
            scratch_types=[
                pltpu.VMEM((B,), jnp.int32),
                pltpu.VMEM((B,), jnp.int32),
                pltpu.VMEM((B,), jnp.int32),
                pltpu.VMEM((B,), jnp.int32),
                pltpu.VMEM((B,), jnp.int32),
                pltpu.VMEM((B, h), jnp.float32),
                pltpu.VMEM((NR,), jnp.float32),
                pltpu.VMEM((zrows, h), jnp.float32),
                pltpu.SemaphoreType.DMA,
                pltpu.VMEM_SHARED((N, h), jnp.float32),
            ],
        )(body)
        return kfn(ytab, inv, srcv, dstv, typv)

    return run


_sc_agg_64 = _make_sc_agg_kernel(HID)
_sc_agg_32 = _make_sc_agg_kernel(LAB)


@jax.jit
def kernel(edge_index, edge_type, concat_emb, w1, b1, w2, b2,
           rw1, rr1, rb1, rw2, rr2, rb2):
    srcv = edge_index[0].astype(jnp.int32)
    dstv = edge_index[1].astype(jnp.int32)
    typv = edge_type.astype(jnp.int32)

    rw1c = rw1.transpose(1, 0, 2).reshape(EMB, R * HID)
    y1, root1 = _tc_a(concat_emb, w1.T, b1.reshape(1, -1), w2.T,
                      b2.reshape(1, -1), rw1c, rr1, rb1.reshape(1, -1))
    y1 = y1.reshape(NR, HID)

    cnt = _sc_counts(dstv, typv)
    inv = 1.0 / jnp.maximum(cnt[0, :, 0] + cnt[1, :, 0], 1.0)

    p1 = _sc_agg_64(y1, inv, srcv, dstv, typv)

    rw2c = rw2.transpose(1, 0, 2).reshape(HID, R * LAB)
    y2, root2 = _tc_b(root1, p1, rw2c, rr2, rb2.reshape(1, -1))
    y2 = y2.reshape(NR, LAB)

    p2 = _sc_agg_32(y2, inv, srcv, dstv, typv)

    return _tc_f(root2, p2)


# speedup vs baseline: 23.9378x; 1.5715x over previous
import functools

import jax
import jax.numpy as jnp
from jax import lax
from jax.experimental import pallas as pl
from jax.experimental.pallas import tpu as pltpu
from jax.experimental.pallas import tpu_sc as plsc

N = 10000
E = 320000
IN_F = 384
OUT_F = 256
EMB = 128
HID = 64
LAB = 32
R = 8
NR = N * R

L = 16
NC = 2
NS = 16
NW = NC * NS
EW = E // NW
B = 80
NCHUNK = EW // B

NB = 400
NBLK = N // NB


def _tc_a_body(emb, w1t, b1, w2t, b2, rw1c, rr1, rb1, y1, root1):
    x1 = jnp.tanh(jnp.dot(emb[...], w1t[...],
                          preferred_element_type=jnp.float32) + b1[...])
    x2 = jnp.dot(x1, w2t[...], preferred_element_type=jnp.float32) + b2[...]
    y1[...] = jnp.dot(x2, rw1c[...], preferred_element_type=jnp.float32)
    root1[...] = jnp.dot(x2, rr1[...],
                         preferred_element_type=jnp.float32) + rb1[...]


def _tc_b_body(root1, p1, rw2c, rr2, rb2, y2, root2):
    h = jnp.maximum(root1[...] + p1[0] + p1[1], 0.0)
    y2[...] = jnp.dot(h, rw2c[...], preferred_element_type=jnp.float32)
    root2[...] = jnp.dot(h, rr2[...],
                         preferred_element_type=jnp.float32) + rb2[...]


def _tc_f_body(root2, p2, out):
    out[...] = jax.nn.sigmoid(root2[...] + p2[0] + p2[1])


def _full(shape):
    return pl.BlockSpec(shape, lambda i: tuple(0 for _ in shape))


def _tc_a(emb, w1t, b1, w2t, b2, rw1c, rr1, rb1):
    return pl.pallas_call(
        _tc_a_body,
        grid=(NBLK,),
        in_specs=[
            pl.BlockSpec((NB, IN_F), lambda i: (i, 0)),
            _full((IN_F, OUT_F)),
            _full((1, OUT_F)),
            _full((OUT_F, EMB)),
            _full((1, EMB)),
            _full((EMB, R * HID)),
            _full((EMB, HID)),
            _full((1, HID)),
        ],
        out_specs=[
            pl.BlockSpec((NB, R * HID), lambda i: (i, 0)),
            pl.BlockSpec((NB, HID), lambda i: (i, 0)),
        ],
        out_shape=[
            jax.ShapeDtypeStruct((N, R * HID), jnp.float32),
            jax.ShapeDtypeStruct((N, HID), jnp.float32),
        ],
    )(emb, w1t, b1, w2t, b2, rw1c, rr1, rb1)


def _tc_b(root1, p1, rw2c, rr2, rb2):
    return pl.pallas_call(
        _tc_b_body,
        grid=(NBLK,),
        in_specs=[
            pl.BlockSpec((NB, HID), lambda i: (i, 0)),
            pl.BlockSpec((NC, NB, HID), lambda i: (0, i, 0)),
            _full((HID, R * LAB)),
            _full((HID, LAB)),
            _full((1, LAB)),
        ],
        out_specs=[
            pl.BlockSpec((NB, R * LAB), lambda i: (i, 0)),
            pl.BlockSpec((NB, LAB), lambda i: (i, 0)),
        ],
        out_shape=[
            jax.ShapeDtypeStruct((N, R * LAB), jnp.float32),
            jax.ShapeDtypeStruct((N, LAB), jnp.float32),
        ],
    )(root1, p1, rw2c, rr2, rb2)


def _tc_f(root2, p2):
    return pl.pallas_call(
        _tc_f_body,
        grid=(NBLK,),
        in_specs=[
            pl.BlockSpec((NB, LAB), lambda i: (i, 0)),
            pl.BlockSpec((NC, NB, LAB), lambda i: (0, i, 0)),
        ],
        out_specs=pl.BlockSpec((NB, LAB), lambda i: (i, 0)),
        out_shape=jax.ShapeDtypeStruct((N, LAB), jnp.float32),
    )(root2, p2)


def _zrow():
    return jnp.zeros((L,), jnp.float32)


def _onerow():
    return jnp.ones((L,), jnp.float32)


def _splat(v16, lane):
    idx = jnp.full((L, 1), lane, jnp.int32)
    return lax.gather(
        v16, idx,
        lax.GatherDimensionNumbers(
            offset_dims=(), collapsed_slice_dims=(0,), start_index_map=(0,)),
        slice_sizes=(1,),
        mode=lax.GatherScatterMode.PROMISE_IN_BOUNDS)


SB = 400
NSUB = SB // B
NSUP = EW // SB
NGRP = SB // L


def _sc_counts_kernel(pk_hbm, cnt_hbm, p_v, dv2, ones_v, zb_v, sem, acc_sp):
    cid = lax.axis_index("c")
    sid = lax.axis_index("s")
    wid = sid * NC + cid

    def _fill(i, _):
        ones_v[i, :] = _onerow()
        zb_v[i, :] = _zrow()
        return _
    lax.fori_loop(0, B, _fill, None)

    def _zfill(i, _):
        zb_v[i, :] = _zrow()
        return _
    lax.fori_loop(B, 1000, _zfill, None)

    def _zero(j, _):
        pltpu.sync_copy(zb_v, acc_sp.at[pl.ds(sid * (NR // NS) + j * 1000,
                                              1000)])
        return _
    lax.fori_loop(0, NR // NS // 1000, _zero, None)
    plsc.subcore_barrier()

    def _super(i, _):
        pltpu.sync_copy(pk_hbm.at[wid * NSUP + i], p_v)

        def _idx(g, _c):
            sl = pl.ds(g * L, L)
            dv2[g // (B // L), pl.ds((g % (B // L)) * L, L)] = (
                p_v[1, sl] * R + p_v[2, sl])
            return _c
        lax.fori_loop(0, NGRP, _idx, None)
        for k in range(NSUB):
            pltpu.sync_copy(ones_v, acc_sp.at[dv2.at[k]], add=True)
        return _
    lax.fori_loop(0, NSUP, _super, None)
    plsc.subcore_barrier()

    pltpu.sync_copy(acc_sp.at[pl.ds(sid * (NR // NS), NR // NS)],
                    cnt_hbm.at[cid, pl.ds(sid * (NR // NS), NR // NS)])


def _sc_counts(pk):
    kfn = functools.partial(
        pl.kernel,
        out_type=jax.ShapeDtypeStruct((NC, NR, L), jnp.float32),
        mesh=plsc.VectorSubcoreMesh(core_axis_name="c", subcore_axis_name="s"),
        compiler_params=pltpu.CompilerParams(use_tc_tiling_on_sc=False, needs_layout_passes=False),
        scratch_types=[
            pltpu.VMEM((3, SB), jnp.int32),
            pltpu.VMEM((NSUB, B), jnp.int32),
            pltpu.VMEM((B, L), jnp.float32),
            pltpu.VMEM((1000, L), jnp.float32),
            pltpu.SemaphoreType.DMA,
            pltpu.VMEM_SHARED((NR, L), jnp.float32),
        ],
    )(_sc_counts_kernel)
    return kfn(pk)


def _make_sc_agg_kernel(h):
    nsl = h // L
    rows_per_sub = N // NS
    zrows = 25

    def body(ytab_hbm, invt_hbm, pk_hbm, part_hbm,
             p_v, gv2, dv2, wiv2, rows_v, winv_v, zb_v, sem, acc_sp):
        cid = lax.axis_index("c")
        sid = lax.axis_index("s")
        wid = sid * NC + cid

        def _zfill(i, _):
            for k in range(nsl):
                zb_v[i, pl.ds(k * L, L)] = _zrow()
            return _
        lax.fori_loop(0, zrows, _zfill, None)

        def _zero(j, _):
            pltpu.sync_copy(zb_v,
                            acc_sp.at[pl.ds(sid * rows_per_sub + j * zrows,
                                            zrows)])
            return _
        lax.fori_loop(0, rows_per_sub // zrows, _zero, None)
        plsc.subcore_barrier()

        def _super(i, _):
            pltpu.sync_copy(pk_hbm.at[wid * NSUP + i], p_v)

            def _idx(g, _c):
                sl = pl.ds(g * L, L)
                t16 = p_v[2, sl]
                gpb = B // L
                ds2 = pl.ds((g % gpb) * L, L)
                gv2[g // gpb, ds2] = p_v[0, sl] * R + t16
                dv2[g // gpb, ds2] = p_v[1, sl]
                wiv2[g // gpb, ds2] = p_v[1, sl] * R + t16
                return _c
            lax.fori_loop(0, NGRP, _idx, None)

            waits = [
                pltpu.async_copy(ytab_hbm.at[gv2.at[k]],
                                 rows_v.at[pl.ds(k * B, B)], sem)
                for k in range(NSUB)
            ] + [
                pltpu.async_copy(invt_hbm.at[wiv2.at[k]],
                                 winv_v.at[pl.ds(k * B, B)], sem)
                for k in range(NSUB)
            ]
            for w in waits:
                w.wait()

            def _scale(g, _c):
                for j in range(L):
                    e = g * L + j
                    ws = winv_v[e, :]
                    for k in range(nsl):
                        sl = pl.ds(k * L, L)
                        rows_v[e, sl] = rows_v[e, sl] * ws
                return _c
            lax.fori_loop(0, NGRP, _scale, None)

            for k in range(NSUB):
                pltpu.sync_copy(rows_v.at[pl.ds(k * B, B)],
                                acc_sp.at[dv2.at[k]], add=True)
            return _
        lax.fori_loop(0, NSUP, _super, None)
        plsc.subcore_barrier()

        pltpu.sync_copy(acc_sp.at[pl.ds(sid * rows_per_sub, rows_per_sub)],
                        part_hbm.at[cid, pl.ds(sid * rows_per_sub,
                                               rows_per_sub)])

    def run(ytab, inv, pk):
        kfn = functools.partial(
            pl.kernel,
            out_type=jax.ShapeDtypeStruct((NC, N, h), jnp.float32),
            mesh=plsc.VectorSubcoreMesh(core_axis_name="c",
                                        subcore_axis_name="s"),
            compiler_params=pltpu.CompilerParams(use_tc_tiling_on_sc=False, needs_layout_passes=False),
            scratch_types=[
                pltpu.VMEM((3, SB), jnp.int32),
                pltpu.VMEM((NSUB, B), jnp.int32),
                pltpu.VMEM((NSUB, B), jnp.int32),
                pltpu.VMEM((NSUB, B), jnp.int32),
                pltpu.VMEM((SB, h), jnp.float32),
                pltpu.VMEM((SB, L), jnp.float32),
                pltpu.VMEM((zrows, h), jnp.float32),
                pltpu.SemaphoreType.DMA,
                pltpu.VMEM_SHARED((N, h), jnp.float32),
            ],
        )(body)
        return kfn(ytab, inv, pk)

    return run


_sc_agg_64 = _make_sc_agg_kernel(HID)
_sc_agg_32 = _make_sc_agg_kernel(LAB)


@jax.jit
def kernel(edge_index, edge_type, concat_emb, w1, b1, w2, b2,
           rw1, rr1, rb1, rw2, rr2, rb2):
    srcv = edge_index[0].astype(jnp.int32)
    dstv = edge_index[1].astype(jnp.int32)
    typv = edge_type.astype(jnp.int32)
    pk = jnp.stack([srcv.reshape(-1, SB), dstv.reshape(-1, SB),
                    typv.reshape(-1, SB)], axis=1)

    rw1c = rw1.transpose(1, 0, 2).reshape(EMB, R * HID)
    y1, root1 = _tc_a(concat_emb, w1.T, b1.reshape(1, -1), w2.T,
                      b2.reshape(1, -1), rw1c, rr1, rb1.reshape(1, -1))
    y1 = y1.reshape(NR, HID)

    cnt = _sc_counts(pk)
    inv = 1.0 / jnp.maximum(cnt[0] + cnt[1], 1.0)

    p1 = _sc_agg_64(y1, inv, pk)

    rw2c = rw2.transpose(1, 0, 2).reshape(HID, R * LAB)
    y2, root2 = _tc_b(root1, p1, rw2c, rr2, rb2.reshape(1, -1))
    y2 = y2.reshape(NR, LAB)

    p2 = _sc_agg_32(y2, inv, pk)

    return _tc_f(root2, p2)


# speedup vs baseline: 27.6310x; 1.1543x over previous
import functools

import jax
import jax.numpy as jnp
from jax import lax
from jax.experimental import pallas as pl
from jax.experimental.pallas import tpu as pltpu
from jax.experimental.pallas import tpu_sc as plsc

N = 10000
E = 320000
IN_F = 384
OUT_F = 256
EMB = 128
HID = 64
LAB = 32
R = 8
NR = N * R

L = 16
NC = 2
NS = 16
NW = NC * NS
EW = E // NW
B = 80
NCHUNK = EW // B

NB = 400
NBLK = N // NB


def _tc_a_body(emb, w1t, b1, w2t, b2, rw1c, rr1, rb1, y1, root1):
    x1 = jnp.tanh(jnp.dot(emb[...], w1t[...],
                          preferred_element_type=jnp.float32) + b1[...])
    x2 = jnp.dot(x1, w2t[...], preferred_element_type=jnp.float32) + b2[...]
    y1[...] = jnp.dot(x2, rw1c[...], preferred_element_type=jnp.float32)
    root1[...] = jnp.dot(x2, rr1[...],
                         preferred_element_type=jnp.float32) + rb1[...]


def _tc_b_body(root1, p1, rw2c, rr2, rb2, y2, root2):
    h = jnp.maximum(root1[...] + p1[0] + p1[1], 0.0)
    y2[...] = jnp.dot(h, rw2c[...], preferred_element_type=jnp.float32)
    root2[...] = jnp.dot(h, rr2[...],
                         preferred_element_type=jnp.float32) + rb2[...]


def _tc_f_body(root2, p2, out):
    out[...] = jax.nn.sigmoid(root2[...] + p2[0] + p2[1])


def _full(shape):
    return pl.BlockSpec(shape, lambda i: tuple(0 for _ in shape))


def _tc_a(emb, w1t, b1, w2t, b2, rw1c, rr1, rb1):
    return pl.pallas_call(
        _tc_a_body,
        grid=(NBLK,),
        in_specs=[
            pl.BlockSpec((NB, IN_F), lambda i: (i, 0)),
            _full((IN_F, OUT_F)),
            _full((1, OUT_F)),
            _full((OUT_F, EMB)),
            _full((1, EMB)),
            _full((EMB, R * HID)),
            _full((EMB, HID)),
            _full((1, HID)),
        ],
        out_specs=[
            pl.BlockSpec((NB, R * HID), lambda i: (i, 0)),
            pl.BlockSpec((NB, HID), lambda i: (i, 0)),
        ],
        out_shape=[
            jax.ShapeDtypeStruct((N, R * HID), jnp.float32),
            jax.ShapeDtypeStruct((N, HID), jnp.float32),
        ],
    )(emb, w1t, b1, w2t, b2, rw1c, rr1, rb1)


def _tc_b(root1, p1, rw2c, rr2, rb2):
    return pl.pallas_call(
        _tc_b_body,
        grid=(NBLK,),
        in_specs=[
            pl.BlockSpec((NB, HID), lambda i: (i, 0)),
            pl.BlockSpec((NC, NB, HID), lambda i: (0, i, 0)),
            _full((HID, R * LAB)),
            _full((HID, LAB)),
            _full((1, LAB)),
        ],
        out_specs=[
            pl.BlockSpec((NB, R * LAB), lambda i: (i, 0)),
            pl.BlockSpec((NB, LAB), lambda i: (i, 0)),
        ],
        out_shape=[
            jax.ShapeDtypeStruct((N, R * LAB), jnp.float32),
            jax.ShapeDtypeStruct((N, LAB), jnp.float32),
        ],
    )(root1, p1, rw2c, rr2, rb2)


def _tc_f(root2, p2):
    return pl.pallas_call(
        _tc_f_body,
        grid=(NBLK,),
        in_specs=[
            pl.BlockSpec((NB, LAB), lambda i: (i, 0)),
            pl.BlockSpec((NC, NB, LAB), lambda i: (0, i, 0)),
        ],
        out_specs=pl.BlockSpec((NB, LAB), lambda i: (i, 0)),
        out_shape=jax.ShapeDtypeStruct((N, LAB), jnp.float32),
    )(root2, p2)


def _zrow():
    return jnp.zeros((L,), jnp.float32)


def _onerow():
    return jnp.ones((L,), jnp.float32)


def _splat(v16, lane):
    idx = jnp.full((L, 1), lane, jnp.int32)
    return lax.gather(
        v16, idx,
        lax.GatherDimensionNumbers(
            offset_dims=(), collapsed_slice_dims=(0,), start_index_map=(0,)),
        slice_sizes=(1,),
        mode=lax.GatherScatterMode.PROMISE_IN_BOUNDS)


SB = 400
NSUB = SB // B
NSUP = EW // SB
NGRP = SB // L


def _sc_counts_kernel(pk_hbm, cnt_hbm, p_v, dv2, ones_v, zb_v, sem, acc_sp):
    cid = lax.axis_index("c")
    sid = lax.axis_index("s")
    wid = sid * NC + cid

    def _fill(i, _):
        ones_v[i, :] = _onerow()
        zb_v[i, :] = _zrow()
        return _
    lax.fori_loop(0, B, _fill, None)

    def _zfill(i, _):
        zb_v[i, :] = _zrow()
        return _
    lax.fori_loop(B, 1000, _zfill, None)

    def _zero(j, _):
        pltpu.sync_copy(zb_v, acc_sp.at[pl.ds(sid * (NR // NS) + j * 1000,
                                              1000)])
        return _
    lax.fori_loop(0, NR // NS // 1000, _zero, None)
    plsc.subcore_barrier()

    def _super(i, _):
        pltpu.sync_copy(pk_hbm.at[wid * NSUP + i], p_v)

        def _idx(g, _c):
            sl = pl.ds(g * L, L)
            dv2[g // (B // L), pl.ds((g % (B // L)) * L, L)] = (
                p_v[1, sl] * R + p_v[2, sl])
            return _c
        lax.fori_loop(0, NGRP, _idx, None)
        for k in range(NSUB):
            pltpu.sync_copy(ones_v, acc_sp.at[dv2.at[k]], add=True)
        return _
    lax.fori_loop(0, NSUP, _super, None)
    plsc.subcore_barrier()

    pltpu.sync_copy(acc_sp.at[pl.ds(sid * (NR // NS), NR // NS)],
                    cnt_hbm.at[cid, pl.ds(sid * (NR // NS), NR // NS)])


def _sc_counts(pk):
    kfn = functools.partial(
        pl.kernel,
        out_type=jax.ShapeDtypeStruct((NC, NR, L), jnp.float32),
        mesh=plsc.VectorSubcoreMesh(core_axis_name="c", subcore_axis_name="s"),
        compiler_params=pltpu.CompilerParams(use_tc_tiling_on_sc=False, needs_layout_passes=False),
        scratch_types=[
            pltpu.VMEM((3, SB), jnp.int32),
            pltpu.VMEM((NSUB, B), jnp.int32),
            pltpu.VMEM((B, L), jnp.float32),
            pltpu.VMEM((1000, L), jnp.float32),
            pltpu.SemaphoreType.DMA,
            pltpu.VMEM_SHARED((NR, L), jnp.float32),
        ],
    )(_sc_counts_kernel)
    return kfn(pk)


def _make_sc_agg_kernel(h):
    nsl = h // L
    rows_per_sub = N // NS
    zrows = 25

    def body(ytab_hbm, invt_hbm, pk_hbm, part_hbm,
             p_a, gv_a, dv_a, wi_a, rows_a, winv_a,
             p_b, gv_b, dv_b, wi_b, rows_b, winv_b,
             zb_v, sem_a, sem_b, acc_sp):
        cid = lax.axis_index("c")
        sid = lax.axis_index("s")
        wid = sid * NC + cid
        buf_a = (p_a, gv_a, dv_a, wi_a, rows_a, winv_a, sem_a)
        buf_b = (p_b, gv_b, dv_b, wi_b, rows_b, winv_b, sem_b)

        def _zfill(i, _):
            for k in range(nsl):
                zb_v[i, pl.ds(k * L, L)] = _zrow()
            return _
        lax.fori_loop(0, zrows, _zfill, None)

        def _zero(j, _):
            pltpu.sync_copy(zb_v,
                            acc_sp.at[pl.ds(sid * rows_per_sub + j * zrows,
                                            zrows)])
            return _
        lax.fori_loop(0, rows_per_sub // zrows, _zero, None)
        plsc.subcore_barrier()

        def _prefetch(i, buf):
            p_v, gv2, dv2, wiv2, rows_v, winv_v, sem = buf
            pltpu.sync_copy(pk_hbm.at[wid * NSUP + i], p_v)

            def _idx(g, _c):
                sl = pl.ds(g * L, L)
                t16 = p_v[2, sl]
                gpb = B // L
                ds2 = pl.ds((g % gpb) * L, L)
                gv2[g // gpb, ds2] = p_v[0, sl] * R + t16
                dv2[g // gpb, ds2] = p_v[1, sl]
                wiv2[g // gpb, ds2] = p_v[1, sl] * R + t16
                return _c
            lax.fori_loop(0, NGRP, _idx, None)
            for k in range(NSUB):
                pltpu.async_copy(ytab_hbm.at[gv2.at[k]],
                                 rows_v.at[pl.ds(k * B, B)], sem)
                pltpu.async_copy(invt_hbm.at[wiv2.at[k]],
                                 winv_v.at[pl.ds(k * B, B)], sem)

        def _finish(buf):
            p_v, gv2, dv2, wiv2, rows_v, winv_v, sem = buf
            for k in range(NSUB):
                pltpu.make_async_copy(ytab_hbm.at[gv2.at[k]],
                                      rows_v.at[pl.ds(k * B, B)], sem).wait()
                pltpu.make_async_copy(invt_hbm.at[wiv2.at[k]],
                                      winv_v.at[pl.ds(k * B, B)], sem).wait()

            def _scale(g, _c):
                for j in range(L):
                    e = g * L + j
                    ws = winv_v[e, :]
                    for k in range(nsl):
                        sl = pl.ds(k * L, L)
                        rows_v[e, sl] = rows_v[e, sl] * ws
                return _c
            lax.fori_loop(0, NGRP, _scale, None)
            for k in range(NSUB):
                pltpu.sync_copy(rows_v.at[pl.ds(k * B, B)],
                                acc_sp.at[dv2.at[k]], add=True)

        _prefetch(0, buf_a)

        def _pair(i, _):
            _prefetch(2 * i + 1, buf_b)
            _finish(buf_a)
            _prefetch(2 * i + 2, buf_a)
            _finish(buf_b)
            return _
        lax.fori_loop(0, (NSUP - 1) // 2, _pair, None)
        _finish(buf_a)
        plsc.subcore_barrier()

        pltpu.sync_copy(acc_sp.at[pl.ds(sid * rows_per_sub, rows_per_sub)],
                        part_hbm.at[cid, pl.ds(sid * rows_per_sub,
                                               rows_per_sub)])

    def run(ytab, inv, pk):
        kfn = functools.partial(
            pl.kernel,
            out_type=jax.ShapeDtypeStruct((NC, N, h), jnp.float32),
            mesh=plsc.VectorSubcoreMesh(core_axis_name="c",
                                        subcore_axis_name="s"),
            compiler_params=pltpu.CompilerParams(use_tc_tiling_on_sc=False, needs_layout_passes=False),
            scratch_types=(
                2 * [
                    pltpu.VMEM((3, SB), jnp.int32),
                    pltpu.VMEM((NSUB, B), jnp.int32),
                    pltpu.VMEM((NSUB, B), jnp.int32),
                    pltpu.VMEM((NSUB, B), jnp.int32),
                    pltpu.VMEM((SB, h), jnp.float32),
                    pltpu.VMEM((SB, L), jnp.float32),
                ] + [
                    pltpu.VMEM((zrows, h), jnp.float32),
                    pltpu.SemaphoreType.DMA,
                    pltpu.SemaphoreType.DMA,
                    pltpu.VMEM_SHARED((N, h), jnp.float32),
                ]
            ),
        )(body)
        return kfn(ytab, inv, pk)

    return run


_sc_agg_64 = _make_sc_agg_kernel(HID)
_sc_agg_32 = _make_sc_agg_kernel(LAB)


@jax.jit
def kernel(edge_index, edge_type, concat_emb, w1, b1, w2, b2,
           rw1, rr1, rb1, rw2, rr2, rb2):
    srcv = edge_index[0].astype(jnp.int32)
    dstv = edge_index[1].astype(jnp.int32)
    typv = edge_type.astype(jnp.int32)
    pk = jnp.stack([srcv.reshape(-1, SB), dstv.reshape(-1, SB),
                    typv.reshape(-1, SB)], axis=1)

    rw1c = rw1.transpose(1, 0, 2).reshape(EMB, R * HID)
    y1, root1 = _tc_a(concat_emb, w1.T, b1.reshape(1, -1), w2.T,
                      b2.reshape(1, -1), rw1c, rr1, rb1.reshape(1, -1))
    y1 = y1.reshape(NR, HID)

    cnt = _sc_counts(pk)
    inv = 1.0 / jnp.maximum(cnt[0] + cnt[1], 1.0)

    p1 = _sc_agg_64(y1, inv, pk)

    rw2c = rw2.transpose(1, 0, 2).reshape(HID, R * LAB)
    y2, root2 = _tc_b(root1, p1, rw2c, rr2, rb2.reshape(1, -1))
    y2 = y2.reshape(NR, LAB)

    p2 = _sc_agg_32(y2, inv, pk)

    return _tc_f(root2, p2)
